# baseline (speedup 1.0000x reference)
import jax
import jax.numpy as jnp
from jax.experimental import pallas as pl
from jax.experimental.pallas import tpu as pltpu

_BLOCK = 12800


def _fused_kernel(nt_ref, x_ref, wt_ref, q_ref, o_ref):
    xb = x_ref[...]
    nt = nt_ref[0]
    bsz = xb.shape[0]
    n_types = q_ref.shape[0]
    oh_t = (jax.lax.broadcasted_iota(jnp.int32, (n_types, bsz), 0) == nt
            ).astype(jnp.float32)
    acc = jax.lax.dot_general(
        xb, wt_ref[...], (((1,), (0,)), ((), ())),
        preferred_element_type=jnp.float32,
        precision=jax.lax.Precision.DEFAULT)
    acc = acc + jax.lax.dot_general(
        oh_t, q_ref[...], (((0,), (0,)), ((), ())),
        preferred_element_type=jnp.float32,
        precision=jax.lax.Precision.DEFAULT)
    o_ref[...] = acc


def kernel(x, node_type, q_table, b_weight):
    n, d_bits = x.shape
    n_types, d_out = q_table.shape
    bsz = _BLOCK
    nb = pl.cdiv(n, bsz)
    n_pad = nb * bsz
    nt3 = jnp.pad(node_type.astype(jnp.int32), (0, n_pad - n)).reshape(
        nb, 1, bsz)
    wt = b_weight.T
    return pl.pallas_call(
        _fused_kernel,
        grid=(nb,),
        in_specs=[
            pl.BlockSpec((1, 1, bsz), lambda i: (i, 0, 0)),
            pl.BlockSpec((bsz, d_bits), lambda i: (i, 0)),
            pl.BlockSpec((d_bits, d_out), lambda i: (0, 0)),
            pl.BlockSpec((n_types, d_out), lambda i: (0, 0)),
        ],
        out_specs=pl.BlockSpec((bsz, d_out), lambda i: (i, 0)),
        out_shape=jax.ShapeDtypeStruct((n, d_out), jnp.float32),
        compiler_params=pltpu.CompilerParams(
            dimension_semantics=("parallel",)),
    )(nt3, x, wt, q_table)
